# baseline (speedup 1.0000x reference)
import dataclasses
import functools

import jax
import jax.numpy as jnp
from jax import lax
from jax.experimental import pallas as pl
from jax.experimental.pallas import tpu as pltpu
from jax.experimental.pallas import tpu_sc as plsc

_B, _S, _V = 1024, 50, 1000
_B0 = 512
_NW = 32
_BPW = (_B - _B0) // _NW
_CHUNKS = [0, 16, 32, 34]
_BBB = 64
_LW = 128
_TILES = [(t * _LW, min(_LW, _V - t * _LW)) for t in range((_V + _LW - 1) // _LW)]

_sc_mesh = plsc.VectorSubcoreMesh(core_axis_name="c", subcore_axis_name="s")
_sc_cp = pltpu.CompilerParams()
if "needs_layout_passes" in pltpu.CompilerParams.__dataclass_fields__:
    _sc_cp = dataclasses.replace(_sc_cp, needs_layout_passes=False)
if "use_tc_tiling_on_sc" in pltpu.CompilerParams.__dataclass_fields__:
    _sc_cp = dataclasses.replace(_sc_cp, use_tc_tiling_on_sc=True)


@functools.partial(
    pl.kernel,
    compiler_params=_sc_cp,
    out_type=(jax.ShapeDtypeStruct((_NW * 16,), jnp.float32),
              jax.ShapeDtypeStruct((_NW * 16,), jnp.float32)),
    mesh=_sc_mesh,
    scratch_types=[pltpu.VMEM((_S, _V), jnp.float32),
                   pltpu.VMEM((_S, _V), jnp.float32),
                   pltpu.VMEM((_BPW, _S), jnp.int32),
                   pltpu.VMEM((((_BPW + 15) // 16) * 16,), jnp.float32),
                   pltpu.VMEM((((_BPW + 15) // 16) * 16,), jnp.float32),
                   pltpu.VMEM((16,), jnp.float32),
                   pltpu.VMEM((16,), jnp.float32),
                   pltpu.SemaphoreType.DMA,
                   pltpu.SemaphoreType.DMA],
)
def _sc_part(lp_hbm, seq_hbm, rw_hbm, bl_hbm, loss_hbm, cnt_hbm,
             buf0, buf1, seq_v, rw_v, bl_v, lossp_v, cntp_v, sem0, sem1):
    bufs = (buf0, buf1)
    sems = (sem0, sem1)
    wid = lax.axis_index("s") * 2 + lax.axis_index("c")
    b0 = _B0 + wid * _BPW
    pltpu.sync_copy(seq_hbm.at[pl.ds(b0, _BPW)], seq_v)
    pltpu.sync_copy(rw_hbm.at[pl.ds(b0, _BPW)], rw_v.at[pl.ds(0, _BPW)])
    pltpu.sync_copy(bl_hbm.at[pl.ds(b0, _BPW)], bl_v.at[pl.ds(0, _BPW)])
    copies = [pltpu.async_copy(lp_hbm.at[b0], buf0, sem0)]
    acc = jnp.zeros((16,), jnp.float32)
    cnt = jnp.zeros((16,), jnp.float32)
    iota = lax.iota(jnp.int32, 16)
    adv_groups = [rw_v[pl.ds(g * 16, 16)] - bl_v[pl.ds(g * 16, 16)]
                  for g in range((_BPW + 15) // 16)]
    for k in range(_BPW):
        if k + 1 < _BPW:
            copies.append(pltpu.async_copy(
                lp_hbm.at[b0 + k + 1], bufs[(k + 1) % 2],
                sems[(k + 1) % 2]))
        copies[k].wait()
        adv16 = jax.lax.broadcast(
            jnp.sum(jnp.where(iota == k % 16, adv_groups[k // 16], 0.0)),
            (16,))
        slab = bufs[k % 2]
        for c in _CHUNKS:
            s16 = iota + c
            sq16 = seq_v[k, pl.ds(c, 16)]
            valid = s16 < _S
            if c == _CHUNKS[-1]:
                valid = s16 >= _CHUNKS[-2] + 16
            m = valid & (sq16 > 0)
            vals = plsc.load_gather(slab, [s16, sq16], mask=m)
            acc = acc + jnp.where(m, vals * adv16, 0.0)
            cnt = cnt + jnp.where(m, 1.0, 0.0)
    lossp_v[...] = acc
    cntp_v[...] = cnt
    pltpu.sync_copy(lossp_v, loss_hbm.at[pl.ds(wid * 16, 16)])
    pltpu.sync_copy(cntp_v, cnt_hbm.at[pl.ds(wid * 16, 16)])


def _tc_body(reward_ref, baseline_ref, lp_ref, seqs_ref, loss_ref, cnt_ref,
             grand_ref, cntv_ref):
    i = pl.program_id(0)

    @pl.when(i == 0)
    def _init():
        grand_ref[...] = jnp.zeros_like(grand_ref)
        cntv_ref[...] = jnp.zeros_like(cntv_ref)

    for bb in range(_BBB):
        advb = reward_ref[bb, 0] - baseline_ref[bb, 0]
        tgt = seqs_ref[bb]
        pos = tgt > 0
        w = jnp.where(pos, advb, 0.0)
        cntv_ref[...] += pos.astype(jnp.float32)
        for toff, wdt in _TILES:
            iota_t = jax.lax.broadcasted_iota(jnp.int32, (_S, wdt), 1) + toff
            tgt_t = seqs_ref[bb, :, 0:wdt]
            w_t = w[:, 0:wdt]
            eq = tgt_t == iota_t
            lp_t = lp_ref[bb, :, toff:toff + wdt]
            grand_ref[:, toff:toff + wdt] += jnp.where(eq, lp_t * w_t, 0.0)

    @pl.when(i == pl.num_programs(0) - 1)
    def _fin():
        loss_ref[0, 0] = jnp.sum(grand_ref[...])
        cnt_ref[0, 0] = jnp.sum(cntv_ref[...]) * (1.0 / _LW)


def _tc_partial(reward, baseline, log_probs, seq):
    seq_rep = jnp.broadcast_to(seq[:_B0, :, None], (_B0, _S, _LW))
    grid = (_B0 // _BBB,)
    return pl.pallas_call(
        _tc_body,
        grid=grid,
        in_specs=[
            pl.BlockSpec((_BBB, 1), lambda i: (i, 0), memory_space=pltpu.SMEM),
            pl.BlockSpec((_BBB, 1), lambda i: (i, 0), memory_space=pltpu.SMEM),
            pl.BlockSpec((_BBB, _S, _V), lambda i: (i, 0, 0)),
            pl.BlockSpec((_BBB, _S, _LW), lambda i: (i, 0, 0)),
        ],
        out_specs=(pl.BlockSpec(memory_space=pltpu.SMEM),
                   pl.BlockSpec(memory_space=pltpu.SMEM)),
        out_shape=(jax.ShapeDtypeStruct((1, 1), jnp.float32),
                   jax.ShapeDtypeStruct((1, 1), jnp.float32)),
        scratch_shapes=[
            pltpu.VMEM((_S, _V), jnp.float32),
            pltpu.VMEM((_S, _LW), jnp.float32),
        ],
        compiler_params=pltpu.CompilerParams(
            dimension_semantics=("arbitrary",),
        ),
    )(reward, baseline, log_probs, seq_rep)


def _fin_body(scl_ref, scc_ref, tcl_ref, tcc_ref, out_ref):
    loss_sum = -(jnp.sum(scl_ref[...]) + tcl_ref[0, 0])
    cnt = jnp.sum(scc_ref[...]) + tcc_ref[0, 0]
    out_ref[0, 0] = jnp.where(cnt > 0, loss_sum / cnt, loss_sum)


def _finish(sc_loss, sc_cnt, tc_loss, tc_cnt):
    return pl.pallas_call(
        _fin_body,
        in_specs=[
            pl.BlockSpec((4, _NW * 4), lambda: (0, 0)),
            pl.BlockSpec((4, _NW * 4), lambda: (0, 0)),
            pl.BlockSpec(memory_space=pltpu.SMEM),
            pl.BlockSpec(memory_space=pltpu.SMEM),
        ],
        out_specs=pl.BlockSpec(memory_space=pltpu.SMEM),
        out_shape=jax.ShapeDtypeStruct((1, 1), jnp.float32),
    )(sc_loss.reshape(4, _NW * 4), sc_cnt.reshape(4, _NW * 4),
      tc_loss, tc_cnt)


def kernel(reward, baseline, log_probs, seq):
    rw = reward.reshape(-1)
    bl = baseline.reshape(-1)
    sc_loss, sc_cnt = _sc_part(log_probs, seq, rw, bl)
    if _B0 > 0:
        tc_loss, tc_cnt = _tc_partial(reward, baseline, log_probs, seq)
    else:
        tc_loss = jnp.zeros((1, 1), jnp.float32)
        tc_cnt = jnp.zeros((1, 1), jnp.float32)
    out = _finish(sc_loss, sc_cnt, tc_loss, tc_cnt)
    return out[0, 0]


# speedup vs baseline: 1.0501x; 1.0501x over previous
import jax
import jax.numpy as jnp
from jax.experimental import pallas as pl
from jax.experimental.pallas import tpu as pltpu

_B, _S, _V = 1024, 50, 1000
_BBB = 64
_LW = 128
_TILES = [(t * _LW, min(_LW, _V - t * _LW)) for t in range((_V + _LW - 1) // _LW)]


def _tc_body(reward_ref, baseline_ref, lp_ref, seqs_ref, out_ref,
             grand_ref, cnt_ref):
    i = pl.program_id(0)

    @pl.when(i == 0)
    def _init():
        grand_ref[...] = jnp.zeros_like(grand_ref)
        cnt_ref[...] = jnp.zeros_like(cnt_ref)

    for bb in range(_BBB):
        advb = reward_ref[bb, 0] - baseline_ref[bb, 0]
        tgt = seqs_ref[bb]
        pos = tgt > 0
        w = jnp.where(pos, advb, 0.0)
        cnt_ref[...] += pos.astype(jnp.float32)
        for toff, wdt in _TILES:
            iota_t = jax.lax.broadcasted_iota(jnp.int32, (_S, wdt), 1) + toff
            tgt_t = seqs_ref[bb, :, 0:wdt]
            w_t = w[:, 0:wdt]
            eq = tgt_t == iota_t
            lp_t = lp_ref[bb, :, toff:toff + wdt]
            grand_ref[:, toff:toff + wdt] += jnp.where(eq, lp_t * w_t, 0.0)

    @pl.when(i == pl.num_programs(0) - 1)
    def _fin():
        loss_sum = -jnp.sum(grand_ref[...])
        cnt = jnp.sum(cnt_ref[...]) * (1.0 / _LW)
        out_ref[0, 0] = jnp.where(cnt > 0, loss_sum / cnt, loss_sum)


def kernel(reward, baseline, log_probs, seq):
    seq_rep = jnp.broadcast_to(seq[:, :, None], (_B, _S, _LW))
    grid = (_B // _BBB,)
    out = pl.pallas_call(
        _tc_body,
        grid=grid,
        in_specs=[
            pl.BlockSpec((_BBB, 1), lambda i: (i, 0), memory_space=pltpu.SMEM),
            pl.BlockSpec((_BBB, 1), lambda i: (i, 0), memory_space=pltpu.SMEM),
            pl.BlockSpec((_BBB, _S, _V), lambda i: (i, 0, 0)),
            pl.BlockSpec((_BBB, _S, _LW), lambda i: (i, 0, 0)),
        ],
        out_specs=pl.BlockSpec(memory_space=pltpu.SMEM),
        out_shape=jax.ShapeDtypeStruct((1, 1), jnp.float32),
        scratch_shapes=[
            pltpu.VMEM((_S, _V), jnp.float32),
            pltpu.VMEM((_S, _LW), jnp.float32),
        ],
        compiler_params=pltpu.CompilerParams(
            dimension_semantics=("arbitrary",),
        ),
    )(reward, baseline, log_probs, seq_rep)
    return out[0, 0]
